# baseline (speedup 1.0000x reference)
import functools

import jax
import jax.numpy as jnp
from jax import lax
from jax.experimental import pallas as pl
from jax.experimental.pallas import tpu as pltpu
from jax.experimental.pallas import tpu_sc as plsc

HIDDEN = 32
BATCH = 16384

_INFO = plsc.get_sparse_core_info()
NC = _INFO.num_cores
NS = _INFO.num_subcores
LANES = _INFO.num_lanes
NW = NC * NS
B_PER_W = BATCH // NW
CHUNK = 128
NCHUNK = B_PER_W // CHUNK


def _mf_body(uid_hbm, iid_hbm, ut_hbm, it_hbm, out_hbm,
             uidx_v, iidx_v, urows_v, irows_v, out_v, sem):
  wid = lax.axis_index("s") * NC + lax.axis_index("c")
  base = wid * B_PER_W

  for j in range(NCHUNK):
    pltpu.sync_copy(uid_hbm.at[pl.ds(base + j * CHUNK, CHUNK)], uidx_v.at[j])
    pltpu.sync_copy(iid_hbm.at[pl.ds(base + j * CHUNK, CHUNK)], iidx_v.at[j])

  copies = []
  for j in range(NCHUNK):
    copies.append(pltpu.async_copy(
        ut_hbm.at[uidx_v.at[j]], urows_v.at[pl.ds(j * CHUNK, CHUNK)], sem))
    copies.append(pltpu.async_copy(
        it_hbm.at[iidx_v.at[j]], irows_v.at[pl.ds(j * CHUNK, CHUNK)], sem))
  for c in copies:
    c.wait()

  def group(g, carry):
    rows = g * LANES + lax.iota(jnp.int32, LANES)
    acc = jnp.zeros((LANES,), jnp.float32)
    for h in range(HIDDEN):
      hcol = jnp.full((LANES,), h, jnp.int32)
      uc = plsc.load_gather(urows_v, [rows, hcol])
      ic = plsc.load_gather(irows_v, [rows, hcol])
      acc = acc + uc * ic
    out_v[pl.ds(g * LANES, LANES)] = acc
    return carry

  lax.fori_loop(0, B_PER_W // LANES, group, 0)

  pltpu.sync_copy(out_v, out_hbm.at[pl.ds(base, B_PER_W)])


@jax.jit
def _mf(user_ids, item_ids, user_table, item_table):
  mesh = plsc.VectorSubcoreMesh(core_axis_name="c", subcore_axis_name="s")
  kern = pl.kernel(
      _mf_body,
      mesh=mesh,
      out_type=jax.ShapeDtypeStruct((BATCH,), jnp.float32),
      scratch_types=[
          pltpu.VMEM((NCHUNK, CHUNK), jnp.int32),
          pltpu.VMEM((NCHUNK, CHUNK), jnp.int32),
          pltpu.VMEM((B_PER_W, HIDDEN), jnp.float32),
          pltpu.VMEM((B_PER_W, HIDDEN), jnp.float32),
          pltpu.VMEM((B_PER_W,), jnp.float32),
          pltpu.SemaphoreType.DMA,
      ],
      compiler_params=pltpu.CompilerParams(
          needs_layout_passes=False, use_tc_tiling_on_sc=False),
  )
  return kern(user_ids, item_ids, user_table, item_table)


def kernel(user_ids, item_ids, user_table, item_table):
  user_ids = user_ids.astype(jnp.int32)
  item_ids = item_ids.astype(jnp.int32)
  return _mf(user_ids, item_ids, user_table, item_table)


# speedup vs baseline: 1.4641x; 1.4641x over previous
import jax
import jax.numpy as jnp
from jax import lax
from jax.experimental import pallas as pl
from jax.experimental.pallas import tpu as pltpu
from jax.experimental.pallas import tpu_sc as plsc

HIDDEN = 32
BATCH = 16384

_INFO = plsc.get_sparse_core_info()
NC = _INFO.num_cores
NS = _INFO.num_subcores
LANES = _INFO.num_lanes
NW = NC * NS
B_PER_W = BATCH // NW
PASS_IDS = 128
NPASS = B_PER_W // PASS_IDS
GRP_PER_PASS = PASS_IDS // LANES


def _mf_body(uid_hbm, iid_hbm, ut_hbm, it_hbm, out_hbm,
             uids_v, iids_v, urows_v, irows_v, out_v, sem):
  wid = lax.axis_index("s") * NC + lax.axis_index("c")
  base = wid * B_PER_W

  pltpu.sync_copy(uid_hbm.at[pl.ds(base, B_PER_W)], uids_v)
  pltpu.sync_copy(iid_hbm.at[pl.ds(base, B_PER_W)], iids_v)

  def fire_group(p, g):
    uidv = uids_v[pl.ds(p * PASS_IDS + g * LANES, LANES)]
    iidv = iids_v[pl.ds(p * PASS_IDS + g * LANES, LANES)]
    for k in range(LANES):
      slot = g * LANES + k
      pltpu.async_copy(ut_hbm.at[pl.ds(uidv[k], 1)],
                       urows_v.at[pl.ds(slot, 1)], sem)
      pltpu.async_copy(it_hbm.at[pl.ds(iidv[k], 1)],
                       irows_v.at[pl.ds(slot, 1)], sem)

  def drain_group():
    for _ in range(2 * LANES):
      pltpu.make_async_copy(ut_hbm.at[pl.ds(0, 1)],
                            urows_v.at[pl.ds(0, 1)], sem).wait()

  def compute_group(p, g):
    rows = g * LANES + lax.iota(jnp.int32, LANES)
    acc = jnp.zeros((LANES,), jnp.float32)
    for h in range(HIDDEN):
      hcol = jnp.full((LANES,), h, jnp.int32)
      uc = plsc.load_gather(urows_v, [rows, hcol])
      ic = plsc.load_gather(irows_v, [rows, hcol])
      acc = acc + uc * ic
    out_v[pl.ds(p * PASS_IDS + g * LANES, LANES)] = acc

  for p in range(NPASS):
    fire_group(p, 0)

    def step(g, carry, p=p):
      fire_group(p, g)
      drain_group()
      return carry

    lax.fori_loop(1, GRP_PER_PASS, step, 0)
    drain_group()

    def cstep(g, carry, p=p):
      compute_group(p, g)
      return carry

    lax.fori_loop(0, GRP_PER_PASS, cstep, 0)

  pltpu.sync_copy(out_v, out_hbm.at[pl.ds(base, B_PER_W)])


@jax.jit
def _mf(user_ids, item_ids, user_table, item_table):
  mesh = plsc.VectorSubcoreMesh(core_axis_name="c", subcore_axis_name="s")
  kern = pl.kernel(
      _mf_body,
      mesh=mesh,
      out_type=jax.ShapeDtypeStruct((BATCH,), jnp.float32),
      scratch_types=[
          pltpu.VMEM((B_PER_W,), jnp.int32),
          pltpu.VMEM((B_PER_W,), jnp.int32),
          pltpu.VMEM((PASS_IDS, HIDDEN), jnp.float32),
          pltpu.VMEM((PASS_IDS, HIDDEN), jnp.float32),
          pltpu.VMEM((B_PER_W,), jnp.float32),
          pltpu.SemaphoreType.DMA,
      ],
      compiler_params=pltpu.CompilerParams(needs_layout_passes=False),
  )
  return kern(user_ids, item_ids, user_table, item_table)


def kernel(user_ids, item_ids, user_table, item_table):
  user_ids = user_ids.astype(jnp.int32)
  item_ids = item_ids.astype(jnp.int32)
  return _mf(user_ids, item_ids, user_table, item_table)


# speedup vs baseline: 1.4860x; 1.0149x over previous
import jax
import jax.numpy as jnp
from jax import lax
from jax.experimental import pallas as pl
from jax.experimental.pallas import tpu as pltpu
from jax.experimental.pallas import tpu_sc as plsc

HIDDEN = 32
BATCH = 16384

_INFO = plsc.get_sparse_core_info()
NC = _INFO.num_cores
NS = _INFO.num_subcores
LANES = _INFO.num_lanes
NW = NC * NS
B_PER_W = BATCH // NW
PASS_IDS = 128
NPASS = B_PER_W // PASS_IDS
GRP_PER_PASS = PASS_IDS // LANES


def _mf_body(uid_hbm, iid_hbm, ut_hbm, it_hbm, out_hbm,
             uids_v, iids_v, urows_v, irows_v, out_v, sem):
  wid = lax.axis_index("s") * NC + lax.axis_index("c")
  base = wid * B_PER_W

  pltpu.sync_copy(uid_hbm.at[pl.ds(base, B_PER_W)], uids_v)
  pltpu.sync_copy(iid_hbm.at[pl.ds(base, B_PER_W)], iids_v)

  def fire_group(p, g):
    uidv = uids_v[pl.ds(p * PASS_IDS + g * LANES, LANES)]
    iidv = iids_v[pl.ds(p * PASS_IDS + g * LANES, LANES)]
    for k in range(LANES):
      slot = g * LANES + k
      pltpu.async_copy(ut_hbm.at[pl.ds(uidv[k], 1)],
                       urows_v.at[pl.ds(slot, 1)], sem)
      pltpu.async_copy(it_hbm.at[pl.ds(iidv[k], 1)],
                       irows_v.at[pl.ds(slot, 1)], sem)

  def drain_group():
    for _ in range(2 * LANES):
      pltpu.make_async_copy(ut_hbm.at[pl.ds(0, 1)],
                            urows_v.at[pl.ds(0, 1)], sem).wait()

  def compute_group(p, g):
    rows = g * LANES + lax.iota(jnp.int32, LANES)
    acc = jnp.zeros((LANES,), jnp.float32)
    for h in range(HIDDEN):
      hcol = jnp.full((LANES,), h, jnp.int32)
      uc = plsc.load_gather(urows_v, [rows, hcol])
      ic = plsc.load_gather(irows_v, [rows, hcol])
      acc = acc + uc * ic
    out_v[pl.ds(p * PASS_IDS + g * LANES, LANES)] = acc

  for p in range(NPASS):
    def fstep(g, carry, p=p):
      fire_group(p, g)
      return carry

    lax.fori_loop(0, GRP_PER_PASS, fstep, 0)

    def wstep(_, carry):
      pltpu.make_async_copy(ut_hbm.at[pl.ds(0, 1)],
                            urows_v.at[pl.ds(0, 1)], sem).wait()
      return carry

    lax.fori_loop(0, 2 * PASS_IDS, wstep, 0)

    def cstep(g, carry, p=p):
      compute_group(p, g)
      return carry

    lax.fori_loop(0, GRP_PER_PASS, cstep, 0)

  pltpu.sync_copy(out_v, out_hbm.at[pl.ds(base, B_PER_W)])


@jax.jit
def _mf(user_ids, item_ids, user_table, item_table):
  mesh = plsc.VectorSubcoreMesh(core_axis_name="c", subcore_axis_name="s")
  kern = pl.kernel(
      _mf_body,
      mesh=mesh,
      out_type=jax.ShapeDtypeStruct((BATCH,), jnp.float32),
      scratch_types=[
          pltpu.VMEM((B_PER_W,), jnp.int32),
          pltpu.VMEM((B_PER_W,), jnp.int32),
          pltpu.VMEM((PASS_IDS, HIDDEN), jnp.float32),
          pltpu.VMEM((PASS_IDS, HIDDEN), jnp.float32),
          pltpu.VMEM((B_PER_W,), jnp.float32),
          pltpu.SemaphoreType.DMA,
      ],
      compiler_params=pltpu.CompilerParams(needs_layout_passes=False),
  )
  return kern(user_ids, item_ids, user_table, item_table)


def kernel(user_ids, item_ids, user_table, item_table):
  user_ids = user_ids.astype(jnp.int32)
  item_ids = item_ids.astype(jnp.int32)
  return _mf(user_ids, item_ids, user_table, item_table)
